# Initial kernel scaffold; baseline (speedup 1.0000x reference)
import jax
import jax.numpy as jnp
from jax import lax
from jax.experimental import pallas as pl
from jax.experimental.pallas import tpu as pltpu
from jax.experimental.pallas import tpu_sc as plsc

N = 10000
F = 128
HID = 256
E = 320000

NC, NS, L = 2, 16, 16
NW = NC * NS
WIN = 128
WPT = 80
E_PAD = NW * WPT * WIN
N_PAD = 10240
STRIPE = N_PAD // NS
ZR = 64

_MESH = plsc.VectorSubcoreMesh(core_axis_name="c", subcore_axis_name="s")


def _fill(ref, value):
    rows, cols = ref.shape

    @pl.loop(0, rows)
    def _(i):
        @pl.loop(0, cols, step=L)
        def _(j):
            ref[i, pl.ds(j, L)] = jnp.full((L,), value, jnp.float32)


def _zero_stripe(acc_sh, zero_v, sid):
    _fill(zero_v, 0.0)

    @pl.loop(0, STRIPE, step=ZR)
    def _(r):
        pltpu.sync_copy(zero_v, acc_sh.at[pl.ds(sid * STRIPE + r, ZR)])


def _sc_hist_body(edges_hbm, out_hbm, idx_v, ones_v, zero_v, acc_sh, csem):
    cid = lax.axis_index("c")
    sid = lax.axis_index("s")
    wid = sid * NC + cid
    wbase = wid * WPT

    _zero_stripe(acc_sh, zero_v, sid)
    _fill(ones_v, 1.0)
    plsc.subcore_barrier()

    @pl.loop(0, WPT)
    def _(w):
        pltpu.sync_copy(edges_hbm.at[wbase + w], idx_v)
        pltpu.sync_copy(ones_v, acc_sh.at[idx_v.at[1]], add=True)

    plsc.subcore_barrier()
    pltpu.async_copy(
        acc_sh.at[pl.ds(sid * STRIPE, STRIPE)],
        out_hbm.at[cid, pl.ds(sid * STRIPE, STRIPE)],
        csem,
    ).wait()


def _sc_agg_body(xs_hbm, edges_hbm, out_hbm, idx_v, rows_v, zero_v, acc_sh,
                 gsem0, gsem1, csem):
    cid = lax.axis_index("c")
    sid = lax.axis_index("s")
    wid = sid * NC + cid
    wbase = wid * WPT

    _zero_stripe(acc_sh, zero_v, sid)
    plsc.subcore_barrier()

    sems = (gsem0, gsem1)
    for b in range(2):
        pltpu.sync_copy(edges_hbm.at[wbase + b], idx_v.at[b])
        pltpu.async_copy(xs_hbm.at[idx_v.at[b, 0]], rows_v.at[b], sems[b])

    @pl.loop(0, WPT, step=2)
    def _(w):
        for b in range(2):
            cur = w + b
            pltpu.make_async_copy(
                xs_hbm.at[idx_v.at[b, 0]], rows_v.at[b], sems[b]
            ).wait()
            pltpu.sync_copy(rows_v.at[b], acc_sh.at[idx_v.at[b, 1]], add=True)

            @pl.when(cur + 2 < WPT)
            def _():
                pltpu.sync_copy(edges_hbm.at[wbase + cur + 2], idx_v.at[b])
                pltpu.async_copy(xs_hbm.at[idx_v.at[b, 0]], rows_v.at[b],
                                 sems[b])

    plsc.subcore_barrier()
    pltpu.async_copy(
        acc_sh.at[pl.ds(sid * STRIPE, STRIPE)],
        out_hbm.at[cid, pl.ds(sid * STRIPE, STRIPE)],
        csem,
    ).wait()


@jax.jit
def _sc_hist(edges):
    kern = pl.kernel(
        _sc_hist_body,
        out_type=jax.ShapeDtypeStruct((NC, N_PAD, L), jnp.float32),
        mesh=_MESH,
        scratch_types=[
            pltpu.VMEM((2, WIN), jnp.int32),
            pltpu.VMEM((WIN, L), jnp.float32),
            pltpu.VMEM((ZR, L), jnp.float32),
            pltpu.VMEM_SHARED((N_PAD, L), jnp.float32),
            pltpu.SemaphoreType.DMA,
        ],
    )
    return kern(edges)


@jax.jit
def _sc_agg(xs, edges):
    kern = pl.kernel(
        _sc_agg_body,
        out_type=jax.ShapeDtypeStruct((NC, N_PAD, F), jnp.float32),
        mesh=_MESH,
        scratch_types=[
            pltpu.VMEM((2, 2, WIN), jnp.int32),
            pltpu.VMEM((2, WIN, F), jnp.float32),
            pltpu.VMEM((ZR, F), jnp.float32),
            pltpu.VMEM_SHARED((N_PAD, F), jnp.float32),
            pltpu.SemaphoreType.DMA,
            pltpu.SemaphoreType.DMA,
            pltpu.SemaphoreType.DMA,
        ],
    )
    return kern(xs, edges)


R_BLK = 1280
_GRID = N_PAD // R_BLK


def _dinv(hist_ref):
    deg = hist_ref[0, :, 0:1] + hist_ref[1, :, 0:1] + 1.0
    return lax.rsqrt(deg)


def _tc_scale_body(hist_ref, x_ref, xs_ref):
    xs_ref[...] = _dinv(hist_ref) * x_ref[...]


def _tc_mid_body(hist_ref, acc_ref, xs_ref, w1_ref, b1_ref, w2_ref, zs_ref):
    dinv = _dinv(hist_ref)
    t = dinv * (acc_ref[0] + acc_ref[1] + xs_ref[...])
    h = jnp.dot(t, w1_ref[...], preferred_element_type=jnp.float32)
    h = jnp.maximum(h + b1_ref[...], 0.0)
    z = jnp.dot(h, w2_ref[...], preferred_element_type=jnp.float32)
    zs_ref[...] = dinv * z


def _tc_final_body(hist_ref, acc_ref, zs_ref, b2_ref, out_ref):
    dinv = _dinv(hist_ref)
    out_ref[...] = dinv * (acc_ref[0] + acc_ref[1] + zs_ref[...]) + b2_ref[...]


def _hist_spec():
    return pl.BlockSpec((NC, R_BLK, L), lambda i: (0, i, 0))


def _acc_spec():
    return pl.BlockSpec((NC, R_BLK, F), lambda i: (0, i, 0))


def _row_spec(width):
    return pl.BlockSpec((R_BLK, width), lambda i: (i, 0))


def _full_spec(shape):
    return pl.BlockSpec(shape, lambda i: tuple(0 for _ in shape))


@jax.jit
def _tc_scale(hist, x_pad):
    return pl.pallas_call(
        _tc_scale_body,
        grid=(_GRID,),
        in_specs=[_hist_spec(), _row_spec(F)],
        out_specs=_row_spec(F),
        out_shape=jax.ShapeDtypeStruct((N_PAD, F), jnp.float32),
    )(hist, x_pad)


@jax.jit
def _tc_mid(hist, acc, xs, w1, b1, w2):
    return pl.pallas_call(
        _tc_mid_body,
        grid=(_GRID,),
        in_specs=[
            _hist_spec(),
            _acc_spec(),
            _row_spec(F),
            _full_spec((F, HID)),
            _full_spec((1, HID)),
            _full_spec((HID, F)),
        ],
        out_specs=_row_spec(F),
        out_shape=jax.ShapeDtypeStruct((N_PAD, F), jnp.float32),
    )(hist, acc, xs, w1, b1, w2)


@jax.jit
def _tc_final(hist, acc, zs, b2):
    return pl.pallas_call(
        _tc_final_body,
        grid=(_GRID,),
        in_specs=[
            _hist_spec(),
            _acc_spec(),
            _row_spec(F),
            _full_spec((1, F)),
        ],
        out_specs=_row_spec(F),
        out_shape=jax.ShapeDtypeStruct((N_PAD, F), jnp.float32),
    )(hist, acc, zs, b2)


def kernel(x, edge_index, W1, b1, W2, b2):
    src = edge_index[0].astype(jnp.int32)
    dst = edge_index[1].astype(jnp.int32)
    pad = E_PAD - E
    src_p = jnp.concatenate([src, jnp.full((pad,), N, jnp.int32)])
    dst_p = jnp.concatenate([dst, jnp.full((pad,), N, jnp.int32)])
    edges = (
        jnp.stack([src_p, dst_p], axis=0)
        .reshape(2, NW * WPT, WIN)
        .transpose(1, 0, 2)
    )
    x_pad = jnp.concatenate(
        [x, jnp.zeros((N_PAD - N, F), jnp.float32)], axis=0
    )

    hist = _sc_hist(edges)
    xs = _tc_scale(hist, x_pad)
    acc1 = _sc_agg(xs, edges)
    zs = _tc_mid(hist, acc1, xs, W1, b1.reshape(1, HID), W2)
    acc2 = _sc_agg(zs, edges)
    out = _tc_final(hist, acc2, zs, b2.reshape(1, F))
    return out[:N]


# speedup vs baseline: 10.2079x; 10.2079x over previous
import jax
import jax.numpy as jnp
from jax import lax
from jax.experimental import pallas as pl
from jax.experimental.pallas import tpu as pltpu
from jax.experimental.pallas import tpu_sc as plsc

N = 10000
F = 128
HID = 256
E = 320000

NC, NS, L = 2, 16, 16
NW = NC * NS
WIN = 128
WPT = 80
E_PAD = NW * WPT * WIN
N_PAD = 10240
STRIPE = N_PAD // NS
ZR = 64

def _mesh():
    return plsc.VectorSubcoreMesh(core_axis_name="c", subcore_axis_name="s")


def _fill(ref, value):
    rows, cols = ref.shape

    @pl.loop(0, rows)
    def _(i):
        @pl.loop(0, cols, step=L)
        def _(j):
            ref[i, pl.ds(j, L)] = jnp.full((L,), value, jnp.float32)


def _zero_stripe(acc_sh, zero_v, sid):
    _fill(zero_v, 0.0)

    @pl.loop(0, STRIPE, step=ZR)
    def _(r):
        pltpu.sync_copy(zero_v, acc_sh.at[pl.ds(sid * STRIPE + r, ZR)])


def _sc_hist_body(edges_hbm, out_hbm, idx_v, ones_v, zero_v, acc_sh, csem):
    cid = lax.axis_index("c")
    sid = lax.axis_index("s")
    wid = sid * NC + cid
    wbase = wid * WPT

    _zero_stripe(acc_sh, zero_v, sid)
    _fill(ones_v, 1.0)
    plsc.subcore_barrier()

    @pl.loop(0, WPT)
    def _(w):
        pltpu.sync_copy(edges_hbm.at[wbase + w], idx_v)
        pltpu.sync_copy(ones_v, acc_sh.at[idx_v.at[1]], add=True)

    plsc.subcore_barrier()
    pltpu.async_copy(
        acc_sh.at[pl.ds(sid * STRIPE, STRIPE)],
        out_hbm.at[cid, pl.ds(sid * STRIPE, STRIPE)],
        csem,
    ).wait()


def _sc_agg_body(xs_hbm, edges_hbm, out_hbm, idx_v, rows_v, zero_v, acc_sh,
                 gsem0, gsem1, csem):
    cid = lax.axis_index("c")
    sid = lax.axis_index("s")
    wid = sid * NC + cid
    wbase = wid * WPT

    _zero_stripe(acc_sh, zero_v, sid)
    plsc.subcore_barrier()

    sems = (gsem0, gsem1)
    for b in range(2):
        pltpu.sync_copy(edges_hbm.at[wbase + b], idx_v.at[b])
        pltpu.async_copy(xs_hbm.at[idx_v.at[b, 0]], rows_v.at[b], sems[b])

    @pl.loop(0, WPT, step=2)
    def _(w):
        for b in range(2):
            cur = w + b
            pltpu.make_async_copy(
                xs_hbm.at[idx_v.at[b, 0]], rows_v.at[b], sems[b]
            ).wait()
            pltpu.sync_copy(rows_v.at[b], acc_sh.at[idx_v.at[b, 1]], add=True)

            @pl.when(cur + 2 < WPT)
            def _():
                pltpu.sync_copy(edges_hbm.at[wbase + cur + 2], idx_v.at[b])
                pltpu.async_copy(xs_hbm.at[idx_v.at[b, 0]], rows_v.at[b],
                                 sems[b])

    plsc.subcore_barrier()
    pltpu.async_copy(
        acc_sh.at[pl.ds(sid * STRIPE, STRIPE)],
        out_hbm.at[cid, pl.ds(sid * STRIPE, STRIPE)],
        csem,
    ).wait()


@jax.jit
def _sc_hist(edges):
    kern = pl.kernel(
        _sc_hist_body,
        out_type=jax.ShapeDtypeStruct((NC, N_PAD, L), jnp.float32),
        mesh=_mesh(),
        scratch_types=[
            pltpu.VMEM((2, WIN), jnp.int32),
            pltpu.VMEM((WIN, L), jnp.float32),
            pltpu.VMEM((ZR, L), jnp.float32),
            pltpu.VMEM_SHARED((N_PAD, L), jnp.float32),
            pltpu.SemaphoreType.DMA,
        ],
    )
    return kern(edges)


@jax.jit
def _sc_agg(xs, edges):
    kern = pl.kernel(
        _sc_agg_body,
        out_type=jax.ShapeDtypeStruct((NC, N_PAD, F), jnp.float32),
        mesh=_mesh(),
        scratch_types=[
            pltpu.VMEM((2, 2, WIN), jnp.int32),
            pltpu.VMEM((2, WIN, F), jnp.float32),
            pltpu.VMEM((ZR, F), jnp.float32),
            pltpu.VMEM_SHARED((N_PAD, F), jnp.float32),
            pltpu.SemaphoreType.DMA,
            pltpu.SemaphoreType.DMA,
            pltpu.SemaphoreType.DMA,
        ],
    )
    return kern(xs, edges)


R_BLK = 1280
_GRID = N_PAD // R_BLK


def _dinv(hist_ref):
    deg = hist_ref[0, :, 0:1] + hist_ref[1, :, 0:1] + 1.0
    return lax.rsqrt(deg)


def _tc_scale_body(hist_ref, x_ref, xs_ref):
    xs_ref[...] = _dinv(hist_ref) * x_ref[...]


def _tc_mid_body(hist_ref, acc_ref, xs_ref, w1_ref, b1_ref, w2_ref, zs_ref):
    dinv = _dinv(hist_ref)
    t = dinv * (acc_ref[0] + acc_ref[1] + xs_ref[...])
    h = jnp.dot(t, w1_ref[...], preferred_element_type=jnp.float32)
    h = jnp.maximum(h + b1_ref[...], 0.0)
    z = jnp.dot(h, w2_ref[...], preferred_element_type=jnp.float32)
    zs_ref[...] = dinv * z


def _tc_final_body(hist_ref, acc_ref, zs_ref, b2_ref, out_ref):
    dinv = _dinv(hist_ref)
    out_ref[...] = dinv * (acc_ref[0] + acc_ref[1] + zs_ref[...]) + b2_ref[...]


def _hist_spec():
    return pl.BlockSpec((NC, R_BLK, L), lambda i: (0, i, 0))


def _acc_spec():
    return pl.BlockSpec((NC, R_BLK, F), lambda i: (0, i, 0))


def _row_spec(width):
    return pl.BlockSpec((R_BLK, width), lambda i: (i, 0))


def _full_spec(shape):
    return pl.BlockSpec(shape, lambda i: tuple(0 for _ in shape))


@jax.jit
def _tc_scale(hist, x_pad):
    return pl.pallas_call(
        _tc_scale_body,
        grid=(_GRID,),
        in_specs=[_hist_spec(), _row_spec(F)],
        out_specs=_row_spec(F),
        out_shape=jax.ShapeDtypeStruct((N_PAD, F), jnp.float32),
    )(hist, x_pad)


@jax.jit
def _tc_mid(hist, acc, xs, w1, b1, w2):
    return pl.pallas_call(
        _tc_mid_body,
        grid=(_GRID,),
        in_specs=[
            _hist_spec(),
            _acc_spec(),
            _row_spec(F),
            _full_spec((F, HID)),
            _full_spec((1, HID)),
            _full_spec((HID, F)),
        ],
        out_specs=_row_spec(F),
        out_shape=jax.ShapeDtypeStruct((N_PAD, F), jnp.float32),
    )(hist, acc, xs, w1, b1, w2)


@jax.jit
def _tc_final(hist, acc, zs, b2):
    return pl.pallas_call(
        _tc_final_body,
        grid=(_GRID,),
        in_specs=[
            _hist_spec(),
            _acc_spec(),
            _row_spec(F),
            _full_spec((1, F)),
        ],
        out_specs=_row_spec(F),
        out_shape=jax.ShapeDtypeStruct((N_PAD, F), jnp.float32),
    )(hist, acc, zs, b2)


def kernel(x, edge_index, W1, b1, W2, b2):
    src = edge_index[0].astype(jnp.int32)
    dst = edge_index[1].astype(jnp.int32)
    pad = E_PAD - E
    src_p = jnp.concatenate([src, jnp.full((pad,), N, jnp.int32)])
    dst_p = jnp.concatenate([dst, jnp.full((pad,), N, jnp.int32)])
    edges = (
        jnp.stack([src_p, dst_p], axis=0)
        .reshape(2, NW * WPT, WIN)
        .transpose(1, 0, 2)
    )
    x_pad = jnp.concatenate(
        [x, jnp.zeros((N_PAD - N, F), jnp.float32)], axis=0
    )

    hist = _sc_hist(edges)
    xs = _tc_scale(hist, x_pad)
    acc1 = _sc_agg(xs, edges)
    zs = _tc_mid(hist, acc1, xs, W1, b1.reshape(1, HID), W2)
    acc2 = _sc_agg(zs, edges)
    out = _tc_final(hist, acc2, zs, b2.reshape(1, F))
    return out[:N]


# speedup vs baseline: 31.7857x; 3.1139x over previous
import jax
import jax.numpy as jnp
from jax import lax
from jax.experimental import pallas as pl
from jax.experimental.pallas import tpu as pltpu
from jax.experimental.pallas import tpu_sc as plsc

N = 10000
F = 128
HID = 256
E = 320000

NC, NS, L = 2, 16, 16
NW = NC * NS
WIN = 128
M = 2
NHALF = 2
HWPT = 40
WPT = NHALF * HWPT
E_PAD = NW * WPT * WIN
N_PAD = 10240
STRIPE = N_PAD // NS

def _mesh():
    return plsc.VectorSubcoreMesh(core_axis_name="c", subcore_axis_name="s")


def _fill(ref, value):
    rows, cols = ref.shape

    @pl.loop(0, rows)
    def _(i):
        @pl.loop(0, cols, step=L)
        def _(j):
            ref[i, pl.ds(j, L)] = jnp.full((L,), value, jnp.float32)


def _zero_stripe(acc_sh, zero_v, sid, zsem):
    del zsem
    zr = zero_v.shape[0]

    @pl.loop(0, STRIPE, step=zr)
    def _(r):
        pltpu.sync_copy(zero_v, acc_sh.at[pl.ds(sid * STRIPE + r, zr)])


def _sc_hist_body(dst_hbm, out_hbm, idx_v, zero_v, ones_v, acc_sh, ssem,
                  csem):
    del ssem
    cid = lax.axis_index("c")
    sid = lax.axis_index("s")
    wid = sid * NC + cid

    _fill(zero_v, 0.0)
    _zero_stripe(acc_sh, zero_v, sid, None)
    _fill(ones_v, 1.0)
    plsc.subcore_barrier()

    wbase = wid * WPT

    @pl.loop(0, WPT)
    def _(w):
        pltpu.sync_copy(dst_hbm.at[wbase + w], idx_v)
        pltpu.sync_copy(ones_v, acc_sh.at[idx_v.at[1]], add=True)

    plsc.subcore_barrier()
    pltpu.async_copy(
        acc_sh.at[pl.ds(sid * STRIPE, STRIPE)],
        out_hbm.at[cid, pl.ds(sid * STRIPE, STRIPE)],
        csem,
    ).wait()


def _sc_agg_body(xs_hbm, edges_hbm, out_hbm, idx_v, rows_v, acc_sh,
                 gsems, ssems, csem):
    cid = lax.axis_index("c")
    sid = lax.axis_index("s")
    wid = sid * NC + cid

    pltpu.async_copy(edges_hbm.at[wid, 0], idx_v, csem)
    _fill(rows_v.at[0], 0.0)
    _zero_stripe(acc_sh, rows_v.at[0], sid, ssems.at[0])
    pltpu.make_async_copy(edges_hbm.at[wid, 0], idx_v, csem).wait()
    plsc.subcore_barrier()

    def g_idx(cur):
        return idx_v.at[cur, 0]

    def s_idx(cur):
        return idx_v.at[cur, 1]

    for h in range(NHALF):
        for k in range(min(2, HWPT)):
            pltpu.async_copy(xs_hbm.at[g_idx(k)], rows_v.at[k % M],
                             gsems.at[k % M])

        @pl.loop(0, HWPT, step=M)
        def _(w):
            for k in range(M):
                b = k
                cur = w + k
                g = cur + 2
                bg = (k + 2) % M
                pltpu.make_async_copy(
                    xs_hbm.at[g_idx(cur)], rows_v.at[b], gsems.at[b]
                ).wait()
                pltpu.async_copy(rows_v.at[b], acc_sh.at[s_idx(cur)],
                                 ssems.at[b], add=True)

                @pl.when(g < HWPT)
                def _():
                    @pl.when(g >= M)
                    def _():
                        pltpu.make_async_copy(
                            rows_v.at[bg], acc_sh.at[s_idx(g - M)],
                            ssems.at[bg]
                        ).wait()

                    pltpu.async_copy(xs_hbm.at[g_idx(g)], rows_v.at[bg],
                                     gsems.at[bg])

        for k in range(M):
            cur = HWPT - M + k
            pltpu.make_async_copy(
                rows_v.at[k], acc_sh.at[s_idx(cur)], ssems.at[k]
            ).wait()

        if h + 1 < NHALF:
            pltpu.async_copy(edges_hbm.at[wid, h + 1], idx_v, csem).wait()

    plsc.subcore_barrier()
    pltpu.async_copy(
        acc_sh.at[pl.ds(sid * STRIPE, STRIPE)],
        out_hbm.at[cid, pl.ds(sid * STRIPE, STRIPE)],
        csem,
    ).wait()


@jax.jit
def _sc_hist(dst_tiles):
    kern = pl.kernel(
        _sc_hist_body,
        out_type=jax.ShapeDtypeStruct((NC, N_PAD, L), jnp.float32),
        mesh=_mesh(),
        scratch_types=[
            pltpu.VMEM((2, WIN), jnp.int32),
            pltpu.VMEM((WIN, L), jnp.float32),
            pltpu.VMEM((WIN, L), jnp.float32),
            pltpu.VMEM_SHARED((N_PAD, L), jnp.float32),
            pltpu.SemaphoreType.DMA,
            pltpu.SemaphoreType.DMA,
        ],
    )
    return kern(dst_tiles)


@jax.jit
def _sc_agg(xs, edges):
    kern = pl.kernel(
        _sc_agg_body,
        out_type=jax.ShapeDtypeStruct((NC, N_PAD, F), jnp.float32),
        mesh=_mesh(),
        scratch_types=[
            pltpu.VMEM((HWPT, 2, WIN), jnp.int32),
            pltpu.VMEM((M, WIN, F), jnp.float32),
            pltpu.VMEM_SHARED((N_PAD, F), jnp.float32),
            pltpu.SemaphoreType.DMA((M,)),
            pltpu.SemaphoreType.DMA((M,)),
            pltpu.SemaphoreType.DMA,
        ],
    )
    return kern(xs, edges)


R_BLK = 1280
_GRID = N_PAD // R_BLK


def _dinv(hist_ref):
    deg = hist_ref[0, :, 0:1] + hist_ref[1, :, 0:1] + 1.0
    return lax.rsqrt(deg)


def _tc_scale_body(hist_ref, x_ref, xs_ref):
    xs_ref[...] = _dinv(hist_ref) * x_ref[...]


def _tc_mid_body(hist_ref, acc_ref, xs_ref, w1_ref, b1_ref, w2_ref, zs_ref):
    dinv = _dinv(hist_ref)
    t = dinv * (acc_ref[0] + acc_ref[1] + xs_ref[...])
    h = jnp.dot(t, w1_ref[...], preferred_element_type=jnp.float32)
    h = jnp.maximum(h + b1_ref[...], 0.0)
    z = jnp.dot(h, w2_ref[...], preferred_element_type=jnp.float32)
    zs_ref[...] = dinv * z


def _tc_final_body(hist_ref, acc_ref, zs_ref, b2_ref, out_ref):
    dinv = _dinv(hist_ref)
    out_ref[...] = dinv * (acc_ref[0] + acc_ref[1] + zs_ref[...]) + b2_ref[...]


def _hist_spec():
    return pl.BlockSpec((NC, R_BLK, L), lambda i: (0, i, 0))


def _acc_spec():
    return pl.BlockSpec((NC, R_BLK, F), lambda i: (0, i, 0))


def _row_spec(width):
    return pl.BlockSpec((R_BLK, width), lambda i: (i, 0))


def _full_spec(shape):
    return pl.BlockSpec(shape, lambda i: tuple(0 for _ in shape))


@jax.jit
def _tc_scale(hist, x_pad):
    return pl.pallas_call(
        _tc_scale_body,
        grid=(_GRID,),
        in_specs=[_hist_spec(), _row_spec(F)],
        out_specs=_row_spec(F),
        out_shape=jax.ShapeDtypeStruct((N_PAD, F), jnp.float32),
    )(hist, x_pad)


@jax.jit
def _tc_mid(hist, acc, xs, w1, b1, w2):
    return pl.pallas_call(
        _tc_mid_body,
        grid=(_GRID,),
        in_specs=[
            _hist_spec(),
            _acc_spec(),
            _row_spec(F),
            _full_spec((F, HID)),
            _full_spec((1, HID)),
            _full_spec((HID, F)),
        ],
        out_specs=_row_spec(F),
        out_shape=jax.ShapeDtypeStruct((N_PAD, F), jnp.float32),
    )(hist, acc, xs, w1, b1, w2)


@jax.jit
def _tc_final(hist, acc, zs, b2):
    return pl.pallas_call(
        _tc_final_body,
        grid=(_GRID,),
        in_specs=[
            _hist_spec(),
            _acc_spec(),
            _row_spec(F),
            _full_spec((1, F)),
        ],
        out_specs=_row_spec(F),
        out_shape=jax.ShapeDtypeStruct((N_PAD, F), jnp.float32),
    )(hist, acc, zs, b2)


def kernel(x, edge_index, W1, b1, W2, b2):
    src = edge_index[0].astype(jnp.int32)
    dst = edge_index[1].astype(jnp.int32)
    pad = E_PAD - E
    pad_idx = N + (jnp.arange(pad, dtype=jnp.int32) % (N_PAD - N))
    src_p = jnp.concatenate([src, pad_idx])
    dst_p = jnp.concatenate([dst, pad_idx])
    edges = (
        jnp.stack([src_p, dst_p], axis=0)
        .reshape(2, NW, NHALF, HWPT, WIN)
        .transpose(1, 2, 3, 0, 4)
    )
    x_pad = jnp.concatenate(
        [x, jnp.zeros((N_PAD - N, F), jnp.float32)], axis=0
    )

    hist = _sc_hist(edges.reshape(NW * WPT, 2, WIN))
    xs = _tc_scale(hist, x_pad)
    acc1 = _sc_agg(xs, edges)
    zs = _tc_mid(hist, acc1, xs, W1, b1.reshape(1, HID), W2)
    acc2 = _sc_agg(zs, edges)
    out = _tc_final(hist, acc2, zs, b2.reshape(1, F))
    return out[:N]
